# Initial kernel scaffold; baseline (speedup 1.0000x reference)
import jax
import jax.numpy as jnp
from jax.experimental import pallas as pl


def kernel(node_states, edge_indices, node_type_ids, Wq, bq, Wk, bk, Wv, bv, edge_emb):
    raise NotImplementedError("write your pallas kernel here")


# speedup vs baseline: 4.3203x; 4.3203x over previous
import functools

import jax
import jax.numpy as jnp
from jax import lax
from jax.experimental import pallas as pl
from jax.experimental.pallas import tpu as pltpu
from jax.experimental.pallas import tpu_sc as plsc

_B, _N, _HID, _H, _NREL, _E = 64, 64, 256, 8, 64, 65536
_D = _HID // _H
_NN = _N * _N

_NW = 32
_BPW = _B // _NW
_CH = 8192
_QCAP = 8192 + 16

_CDIM = (((1,), (1,)), ((), ()))


def _proj_body(x_ref, wq_ref, bq_ref, wk_ref, bk_ref, wv_ref, bv_ref, ee_ref,
               q_ref, k_ref, v_ref, qd_ref):
    x = x_ref[...]
    q = lax.dot_general(x, wq_ref[...], _CDIM, preferred_element_type=jnp.float32) + bq_ref[...]
    k = lax.dot_general(x, wk_ref[...], _CDIM, preferred_element_type=jnp.float32) + bk_ref[...]
    v = lax.dot_general(x, wv_ref[...], _CDIM, preferred_element_type=jnp.float32) + bv_ref[...]
    q_ref[...] = q
    k_ref[...] = k
    v_ref[...] = v
    ee = ee_ref[...]
    for h in range(_H):
        sl = slice(h * _D, (h + 1) * _D)
        qd_ref[:, h * _NREL:(h + 1) * _NREL] = lax.dot_general(
            q[:, sl], ee[:, sl], _CDIM, preferred_element_type=jnp.float32)


def _proj(x, wq, bq, wk, bk, wv, bv, ee):
    rb = 512
    grid = (_B * _N // rb,)
    full = lambda shp: pl.BlockSpec(shp, lambda i: (0, 0))
    return pl.pallas_call(
        _proj_body,
        grid=grid,
        in_specs=[
            pl.BlockSpec((rb, _HID), lambda i: (i, 0)),
            full((_HID, _HID)), full((1, _HID)),
            full((_HID, _HID)), full((1, _HID)),
            full((_HID, _HID)), full((1, _HID)),
            full((_NREL, _HID)),
        ],
        out_specs=[
            pl.BlockSpec((rb, _HID), lambda i: (i, 0)),
            pl.BlockSpec((rb, _HID), lambda i: (i, 0)),
            pl.BlockSpec((rb, _HID), lambda i: (i, 0)),
            pl.BlockSpec((rb, _H * _NREL), lambda i: (i, 0)),
        ],
        out_shape=[
            jax.ShapeDtypeStruct((_B * _N, _HID), jnp.float32),
            jax.ShapeDtypeStruct((_B * _N, _HID), jnp.float32),
            jax.ShapeDtypeStruct((_B * _N, _HID), jnp.float32),
            jax.ShapeDtypeStruct((_B * _N, _H * _NREL), jnp.float32),
        ],
    )(x, wq, bq, wk, bk, wv, bv, ee)


def _ekey_body(ei_ref, out_ref):
    eb = ei_ref[0]
    ei = ei_ref[1]
    ej = ei_ref[2]
    er = ei_ref[3]
    out_ref[...] = (eb << 18) | (ei << 12) | (ej << 6) | er


def _ekey(edge_indices):
    ei3 = edge_indices.reshape(4, 512, 128)
    out = pl.pallas_call(
        _ekey_body,
        out_shape=jax.ShapeDtypeStruct((512, 128), jnp.int32),
    )(ei3)
    return out.reshape(_E)


def _sc_body(ekey_hbm, qdot_hbm, corr_hbm, cnt_hbm,
             ek_v, qa_v, qb_v, qd_v, comb_v):
    cid = lax.axis_index("c")
    sid = lax.axis_index("s")
    wid = cid * 16 + sid
    b0 = wid * _BPW
    iota = lax.iota(jnp.int32, 16)
    m9 = iota < 9
    h7 = iota & 7
    i9 = jnp.minimum(iota, 8)
    hq = h7 * _NREL
    is8 = iota == 8
    zf = jnp.zeros((16,), jnp.float32)
    zi = jnp.zeros((16,), jnp.int32)

    n0 = jnp.int32(0)
    n1 = jnp.int32(0)
    for c in range(_E // _CH):
        pltpu.sync_copy(ekey_hbm.at[pl.ds(c * _CH, _CH)], ek_v)

        def scan_body(g, carry):
            na, nb = carry
            key = ek_v[pl.ds(g * 16, 16)]
            eb = key >> 18
            k18 = key & 0x3FFFF
            ma = eb == b0
            mb = eb == (b0 + 1)
            csa = plsc.cumsum(ma.astype(jnp.int32))
            csb = plsc.cumsum(mb.astype(jnp.int32))
            plsc.store_scatter(qa_v, [csa + (na - 1)], k18, mask=ma)
            plsc.store_scatter(qb_v, [csb + (nb - 1)], k18, mask=mb)
            return na + csa[15], nb + csb[15]

        n0, n1 = lax.fori_loop(0, _CH // 16, scan_body, (n0, n1))

    qa_v[pl.ds(n0, 16)] = zi
    qb_v[pl.ds(n1, 16)] = zi

    for bi, (q_v, nq) in enumerate(((qa_v, n0), (qb_v, n1))):
        b = b0 + bi
        pltpu.sync_copy(qdot_hbm.at[pl.ds(b * _N, _N)], qd_v)
        for hh in range(_H + 1):
            def zero_comb(z, _, hh=hh):
                comb_v[hh, pl.ds(z * 16, 16)] = zf
                return 0
            lax.fori_loop(0, _NN // 16, zero_comb, 0)

        def group_body(g, _, q_v=q_v, nq=nq):
            key16 = q_v[pl.ds(g * 16, 16)]
            for l in range(16):
                e = g * 16 + l
                keyb = jnp.full((16,), key16[l], jnp.int32)
                msk = m9 & jnp.full((16,), e < nq, jnp.bool_)
                row = keyb >> 12
                col = (keyb & 63) + hq
                cidx = (row << 6) + ((keyb >> 6) & 63)
                vals = plsc.load_gather(qd_v, [row, col], mask=msk)
                vals = jnp.where(is8, jnp.float32(1.0), vals)
                plsc.addupdate_scatter(comb_v, [i9, cidx], vals, mask=msk)
            return 0

        lax.fori_loop(0, (nq + 15) // 16, group_body, 0)

        pltpu.sync_copy(comb_v.at[pl.ds(0, _H)], corr_hbm.at[b])
        pltpu.sync_copy(comb_v.at[pl.ds(_H, 1)], cnt_hbm.at[b])


def _sc_corr(ekey, qdot):
    mesh = plsc.VectorSubcoreMesh(core_axis_name="c", subcore_axis_name="s")
    f = pl.kernel(
        _sc_body,
        out_type=(
            jax.ShapeDtypeStruct((_B, _H, _NN), jnp.float32),
            jax.ShapeDtypeStruct((_B, 1, _NN), jnp.float32),
        ),
        mesh=mesh,
        compiler_params=pltpu.CompilerParams(needs_layout_passes=False),
        scratch_types=[
            pltpu.VMEM((_CH,), jnp.int32),
            pltpu.VMEM((_QCAP,), jnp.int32),
            pltpu.VMEM((_QCAP,), jnp.int32),
            pltpu.VMEM((_N, _H * _NREL), jnp.float32),
            pltpu.VMEM((_H + 1, _NN), jnp.float32),
        ],
    )
    return f(ekey, qdot)


def _attn_body(q_ref, k_ref, v_ref, corr_ref, cnt_ref, out_ref):
    q = q_ref[0]
    k = k_ref[0]
    v = v_ref[0]
    mask = cnt_ref[0] > 0.0
    scale = jnp.float32(1.0) / jnp.sqrt(jnp.float32(_D))
    neg = jnp.float32(-jnp.inf)
    for h in range(_H):
        sl = slice(h * _D, (h + 1) * _D)
        lg = lax.dot_general(q[:, sl], k[:, sl], _CDIM, preferred_element_type=jnp.float32)
        lg = (lg - corr_ref[0, h]) * scale
        ml = jnp.where(mask, lg, neg)
        m = jnp.max(ml, axis=1, keepdims=True)
        m = jnp.where(jnp.isfinite(m), m, 0.0)
        e = jnp.exp(jnp.where(mask, lg - m, jnp.float32(-1e30)))
        s = jnp.sum(e, axis=1, keepdims=True)
        p = jnp.where(s > 0, e / jnp.where(s > 0, s, 1.0), 0.0)
        out_ref[0, :, sl] = lax.dot_general(
            p, v[:, sl], (((1,), (0,)), ((), ())), preferred_element_type=jnp.float32)


def _attn(q, k, v, corr, cnt):
    return pl.pallas_call(
        _attn_body,
        grid=(_B,),
        in_specs=[
            pl.BlockSpec((1, _N, _HID), lambda b: (b, 0, 0)),
            pl.BlockSpec((1, _N, _HID), lambda b: (b, 0, 0)),
            pl.BlockSpec((1, _N, _HID), lambda b: (b, 0, 0)),
            pl.BlockSpec((1, _H, _N, _N), lambda b: (b, 0, 0, 0)),
            pl.BlockSpec((1, _N, _N), lambda b: (b, 0, 0)),
        ],
        out_specs=pl.BlockSpec((1, _N, _HID), lambda b: (b, 0, 0)),
        out_shape=jax.ShapeDtypeStruct((_B, _N, _HID), jnp.float32),
    )(q, k, v, corr, cnt)


def kernel(node_states, edge_indices, node_type_ids, Wq, bq, Wk, bk, Wv, bv, edge_emb):
    x = node_states.reshape(_B * _N, _HID)
    q, k, v, qdot = _proj(x, Wq, bq.reshape(1, _HID), Wk, bk.reshape(1, _HID),
                          Wv, bv.reshape(1, _HID), edge_emb)
    ekey = _ekey(edge_indices)
    corr, cnt = _sc_corr(ekey, qdot)
    out = _attn(q.reshape(_B, _N, _HID), k.reshape(_B, _N, _HID),
                v.reshape(_B, _N, _HID), corr.reshape(_B, _H, _N, _N),
                cnt.reshape(_B, _N, _N))
    return out


# speedup vs baseline: 6.3183x; 1.4625x over previous
import functools

import jax
import jax.numpy as jnp
from jax import lax
from jax.experimental import pallas as pl
from jax.experimental.pallas import tpu as pltpu
from jax.experimental.pallas import tpu_sc as plsc

_B, _N, _HID, _H, _NREL, _E = 64, 64, 256, 8, 64, 65536
_D = _HID // _H
_NN = _N * _N
_NP = 128

_NW = 32
_BPW = _B // _NW
_CH = 4096
_QCAP = 8192 + 16

_CDIM = (((1,), (1,)), ((), ()))


def _proj_body(x_ref, wq_ref, bq_ref, wk_ref, bk_ref, wv_ref, bv_ref, ee_ref,
               q_ref, k_ref, v_ref, qd_ref):
    x = x_ref[...]
    q = lax.dot_general(x, wq_ref[...], _CDIM, preferred_element_type=jnp.float32) + bq_ref[...]
    k = lax.dot_general(x, wk_ref[...], _CDIM, preferred_element_type=jnp.float32) + bk_ref[...]
    v = lax.dot_general(x, wv_ref[...], _CDIM, preferred_element_type=jnp.float32) + bv_ref[...]
    ee = ee_ref[...]
    nb = q.shape[0] // _N
    for h in range(_H):
        sl = slice(h * _D, (h + 1) * _D)
        qt = q[:, sl].T
        kt = k[:, sl].T
        vt = v[:, sl].T
        for bb in range(nb):
            cl = slice(bb * _N, (bb + 1) * _N)
            q_ref[bb, h] = qt[:, cl]
            k_ref[bb, h] = kt[:, cl]
            v_ref[bb, h] = vt[:, cl]
        qd_ref[:, h * _NREL:(h + 1) * _NREL] = lax.dot_general(
            q[:, sl], ee[:, sl], _CDIM, preferred_element_type=jnp.float32)


def _proj(x, wq, bq, wk, bk, wv, bv, ee):
    rb = 512
    grid = (_B * _N // rb,)
    full = lambda shp: pl.BlockSpec(shp, lambda i: (0, 0))
    hd = pl.BlockSpec((rb // _N, _H, _D, _N), lambda i: (i, 0, 0, 0))
    return pl.pallas_call(
        _proj_body,
        grid=grid,
        in_specs=[
            pl.BlockSpec((rb, _HID), lambda i: (i, 0)),
            full((_HID, _HID)), full((1, _HID)),
            full((_HID, _HID)), full((1, _HID)),
            full((_HID, _HID)), full((1, _HID)),
            full((_NREL, _HID)),
        ],
        out_specs=[
            hd, hd, hd,
            pl.BlockSpec((rb, _H * _NREL), lambda i: (i, 0)),
        ],
        out_shape=[
            jax.ShapeDtypeStruct((_B, _H, _D, _N), jnp.float32),
            jax.ShapeDtypeStruct((_B, _H, _D, _N), jnp.float32),
            jax.ShapeDtypeStruct((_B, _H, _D, _N), jnp.float32),
            jax.ShapeDtypeStruct((_B * _N, _H * _NREL), jnp.float32),
        ],
    )(x, wq, bq, wk, bk, wv, bv, ee)


def _ekey_body(ei_ref, out_ref):
    eb = ei_ref[0]
    ei = ei_ref[1]
    ej = ei_ref[2]
    er = ei_ref[3]
    out_ref[...] = (eb << 18) | (ei << 12) | (ej << 6) | er


def _ekey(edge_indices):
    ei3 = edge_indices.reshape(4, 512, 128)
    out = pl.pallas_call(
        _ekey_body,
        out_shape=jax.ShapeDtypeStruct((512, 128), jnp.int32),
    )(ei3)
    return out.reshape(_E)


def _sc_body(ekey_hbm, qdot_hbm, corr_hbm, cnt_hbm,
             ek_v, qa_v, qb_v, qd_v, comb_v):
    cid = lax.axis_index("c")
    sid = lax.axis_index("s")
    wid = cid * 16 + sid
    b0 = wid * _BPW
    iota = lax.iota(jnp.int32, 16)
    m9 = iota < 9
    h7 = iota & 7
    i9 = jnp.minimum(iota, 8)
    hq = h7 * _NREL
    is8 = iota == 8
    zf = jnp.zeros((16,), jnp.float32)
    zi = jnp.zeros((16,), jnp.int32)

    nav = jnp.zeros((16,), jnp.int32)
    nbv = jnp.zeros((16,), jnp.int32)
    for c in range(_E // _CH):
        pltpu.sync_copy(ekey_hbm.at[pl.ds(c * _CH, _CH)], ek_v)

        def scan_body(g, carry):
            nav, nbv = carry
            key = ek_v[pl.ds(g * 16, 16)]
            eb = key >> 18
            k18 = key & 0x3FFFF
            ma = eb == b0
            mb = eb == (b0 + 1)
            csa = plsc.cumsum(ma.astype(jnp.int32))
            csb = plsc.cumsum(mb.astype(jnp.int32))
            plsc.store_scatter(qa_v, [csa + nav - 1], k18, mask=ma)
            plsc.store_scatter(qb_v, [csb + nbv - 1], k18, mask=mb)
            nav = nav + plsc.all_reduce_population_count(ma)
            nbv = nbv + plsc.all_reduce_population_count(mb)
            return nav, nbv

        nav, nbv = lax.fori_loop(0, _CH // 16, scan_body, (nav, nbv))

    n0 = nav[0]
    n1 = nbv[0]
    qa_v[pl.ds(n0, 16)] = zi
    qb_v[pl.ds(n1, 16)] = zi

    for bi, (q_v, nq) in enumerate(((qa_v, n0), (qb_v, n1))):
        b = b0 + bi
        pltpu.sync_copy(qdot_hbm.at[pl.ds(b * _N, _N)], qd_v)
        for hh in range(_H + 1):
            def zero_comb(z, _, hh=hh):
                comb_v[hh, z, pl.ds(0, 16)] = zf
                comb_v[hh, z, pl.ds(16, 16)] = zf
                comb_v[hh, z, pl.ds(32, 16)] = zf
                comb_v[hh, z, pl.ds(48, 16)] = zf
                return 0
            lax.fori_loop(0, _N, zero_comb, 0)

        def group_body(g, _, q_v=q_v, nq=nq):
            key16 = q_v[pl.ds(g * 16, 16)]
            for l in range(16):
                e = g * 16 + l
                keyb = jnp.full((16,), key16[l], jnp.int32)
                msk = m9 & jnp.full((16,), e < nq, jnp.bool_)
                row = keyb >> 12
                col = (keyb & 63) + hq
                jrow = (keyb >> 6) & 63
                vals = plsc.load_gather(qd_v, [row, col], mask=msk)
                vals = jnp.where(is8, jnp.float32(1.0), vals)
                plsc.addupdate_scatter(comb_v, [i9, row, jrow], vals, mask=msk)
            return 0

        lax.fori_loop(0, (nq + 15) // 16, group_body, 0)

        pltpu.sync_copy(comb_v.at[pl.ds(0, _H)], corr_hbm.at[b])
        pltpu.sync_copy(comb_v.at[pl.ds(_H, 1)], cnt_hbm.at[b])


def _sc_corr(ekey, qdot):
    mesh = plsc.VectorSubcoreMesh(core_axis_name="c", subcore_axis_name="s")
    f = pl.kernel(
        _sc_body,
        out_type=(
            jax.ShapeDtypeStruct((_B, _H, _N, _NP), jnp.float32),
            jax.ShapeDtypeStruct((_B, 1, _N, _NP), jnp.float32),
        ),
        mesh=mesh,
        compiler_params=pltpu.CompilerParams(needs_layout_passes=False),
        scratch_types=[
            pltpu.VMEM((_CH,), jnp.int32),
            pltpu.VMEM((_QCAP,), jnp.int32),
            pltpu.VMEM((_QCAP,), jnp.int32),
            pltpu.VMEM((_N, _H * _NREL), jnp.float32),
            pltpu.VMEM((_H + 1, _N, _NP), jnp.float32),
        ],
    )
    return f(ekey, qdot)


def _attn_body(q_ref, k_ref, v_ref, corr_ref, cnt_ref, out_ref):
    qh = q_ref[0]
    kh = k_ref[0]
    vh = v_ref[0]
    corr = corr_ref[0][:, :, :_N]
    mask = (cnt_ref[0, 0][:, :_N] > 0.0)[None, :, :]
    scale = jnp.float32(1.0) / jnp.sqrt(jnp.float32(_D))
    neg = jnp.float32(-jnp.inf)
    lg = lax.dot_general(qh, kh, (((1,), (1,)), ((0,), (0,))),
                         preferred_element_type=jnp.float32)
    lg = (lg - corr) * scale
    ml = jnp.where(mask, lg, neg)
    m = jnp.max(ml, axis=2, keepdims=True)
    m = jnp.where(jnp.isfinite(m), m, 0.0)
    e = jnp.exp(jnp.where(mask, lg - m, jnp.float32(-1e30)))
    s = jnp.sum(e, axis=2, keepdims=True)
    p = jnp.where(s > 0, e / jnp.where(s > 0, s, 1.0), 0.0)
    o = lax.dot_general(p, vh, (((2,), (2,)), ((0,), (0,))),
                        preferred_element_type=jnp.float32)
    for h in range(_H):
        out_ref[0, :, h * _D:(h + 1) * _D] = o[h]


def _attn(q, k, v, corr, cnt):
    hd = pl.BlockSpec((1, _H, _D, _N), lambda b: (b, 0, 0, 0))
    return pl.pallas_call(
        _attn_body,
        grid=(_B,),
        in_specs=[
            hd, hd, hd,
            pl.BlockSpec((1, _H, _N, _NP), lambda b: (b, 0, 0, 0)),
            pl.BlockSpec((1, 1, _N, _NP), lambda b: (b, 0, 0, 0)),
        ],
        out_specs=pl.BlockSpec((1, _N, _HID), lambda b: (b, 0, 0)),
        out_shape=jax.ShapeDtypeStruct((_B, _N, _HID), jnp.float32),
    )(q, k, v, corr, cnt)


def kernel(node_states, edge_indices, node_type_ids, Wq, bq, Wk, bk, Wv, bv, edge_emb):
    x = node_states.reshape(_B * _N, _HID)
    q, k, v, qdot = _proj(x, Wq, bq.reshape(1, _HID), Wk, bk.reshape(1, _HID),
                          Wv, bv.reshape(1, _HID), edge_emb)
    ekey = _ekey(edge_indices)
    corr, cnt = _sc_corr(ekey, qdot)
    out = _attn(q, k, v, corr, cnt)
    return out


# speedup vs baseline: 6.7878x; 1.0743x over previous
import functools

import jax
import jax.numpy as jnp
from jax import lax
from jax.experimental import pallas as pl
from jax.experimental.pallas import tpu as pltpu
from jax.experimental.pallas import tpu_sc as plsc

_B, _N, _HID, _H, _NREL, _E = 64, 64, 256, 8, 64, 65536
_D = _HID // _H
_NN = _N * _N
_NP = 128

_NW = 32
_BPW = _B // _NW
_CH = 8192
_QCAP = 8192 + 16

_CDIM = (((1,), (1,)), ((), ()))


def _proj_body(x_ref, wq_ref, bq_ref, wk_ref, bk_ref, wv_ref, bv_ref, ee_ref,
               q_ref, k_ref, v_ref, qd_ref):
    x = x_ref[...]
    q = lax.dot_general(x, wq_ref[...], _CDIM, preferred_element_type=jnp.float32) + bq_ref[...]
    k = lax.dot_general(x, wk_ref[...], _CDIM, preferred_element_type=jnp.float32) + bk_ref[...]
    v = lax.dot_general(x, wv_ref[...], _CDIM, preferred_element_type=jnp.float32) + bv_ref[...]
    ee = ee_ref[...]
    nb = q.shape[0] // _N
    for h in range(_H):
        sl = slice(h * _D, (h + 1) * _D)
        for bb in range(nb):
            rl = slice(bb * _N, (bb + 1) * _N)
            q_ref[bb, h] = q[rl, sl]
            k_ref[bb, h] = k[rl, sl]
            v_ref[bb, h] = v[rl, sl]
        qd_ref[:, h * _NREL:(h + 1) * _NREL] = lax.dot_general(
            q[:, sl], ee[:, sl], _CDIM, preferred_element_type=jnp.float32)


def _proj(x, wq, bq, wk, bk, wv, bv, ee):
    rb = 512
    grid = (_B * _N // rb,)
    full = lambda shp: pl.BlockSpec(shp, lambda i: (0, 0))
    hd = pl.BlockSpec((rb // _N, _H, _N, _D), lambda i: (i, 0, 0, 0))
    return pl.pallas_call(
        _proj_body,
        grid=grid,
        in_specs=[
            pl.BlockSpec((rb, _HID), lambda i: (i, 0)),
            full((_HID, _HID)), full((1, _HID)),
            full((_HID, _HID)), full((1, _HID)),
            full((_HID, _HID)), full((1, _HID)),
            full((_NREL, _HID)),
        ],
        out_specs=[
            hd, hd, hd,
            pl.BlockSpec((rb, _H * _NREL), lambda i: (i, 0)),
        ],
        out_shape=[
            jax.ShapeDtypeStruct((_B, _H, _N, _D), jnp.float32),
            jax.ShapeDtypeStruct((_B, _H, _N, _D), jnp.float32),
            jax.ShapeDtypeStruct((_B, _H, _N, _D), jnp.float32),
            jax.ShapeDtypeStruct((_B * _N, _H * _NREL), jnp.float32),
        ],
    )(x, wq, bq, wk, bk, wv, bv, ee)


def _ekey_body(ei_ref, out_ref):
    eb = ei_ref[0]
    ei = ei_ref[1]
    ej = ei_ref[2]
    er = ei_ref[3]
    out_ref[...] = (eb << 18) | (ei << 12) | (ej << 6) | er


def _ekey(edge_indices):
    ei3 = edge_indices.reshape(4, 512, 128)
    out = pl.pallas_call(
        _ekey_body,
        out_shape=jax.ShapeDtypeStruct((512, 128), jnp.int32),
    )(ei3)
    return out.reshape(_E)


def _sc_body(ekey_hbm, qdot_hbm, corr_hbm, cnt_hbm,
             ek_v, q_v, qd_v, comb_v):
    cid = lax.axis_index("c")
    sid = lax.axis_index("s")
    wid = cid * 16 + sid
    b0 = wid * _BPW
    iota = lax.iota(jnp.int32, 16)
    m9 = iota < 9
    h7 = iota & 7
    i9 = jnp.minimum(iota, 8)
    hq = h7 * _NREL
    is8 = iota == 8
    one = jnp.full((16,), 1, jnp.int32)
    zf = jnp.zeros((16,), jnp.float32)
    zi = jnp.zeros((16,), jnp.int32)

    nav = jnp.zeros((16,), jnp.int32)
    for c in range(_E // _CH):
        pltpu.sync_copy(ekey_hbm.at[pl.ds(c * _CH, _CH)], ek_v)

        def scan_body(t, nav):
            for u in range(4):
                key = ek_v[pl.ds(t * 64 + u * 16, 16)]
                match = (key >> 19) == wid
                cs = plsc.cumsum(match.astype(jnp.int32))
                plsc.store_scatter(q_v, [cs + (nav - one)], key, mask=match)
                nav = nav + plsc.all_reduce_population_count(match)
            return nav

        nav = lax.fori_loop(0, _CH // 64, scan_body, nav)

    nq = nav[0]
    q_v[pl.ds(nq, 16)] = zi

    for bi in range(_BPW):
        b = b0 + bi
        pltpu.sync_copy(qdot_hbm.at[pl.ds(b * _N, _N)], qd_v)
        for hh in range(_H + 1):
            def zero_comb(z, _, hh=hh):
                comb_v[hh, z, pl.ds(0, 16)] = zf
                comb_v[hh, z, pl.ds(16, 16)] = zf
                comb_v[hh, z, pl.ds(32, 16)] = zf
                comb_v[hh, z, pl.ds(48, 16)] = zf
                return 0
            lax.fori_loop(0, _N, zero_comb, 0)

        def group_body(g, _, b=b):
            key16 = q_v[pl.ds(g * 16, 16)]
            for l in range(16):
                e = g * 16 + l
                keyb = jnp.full((16,), key16[l], jnp.int32)
                msk = m9 & jnp.full((16,), e < nq, jnp.bool_) & ((keyb >> 18) == b)
                row = (keyb >> 12) & 63
                col = (keyb & 63) + hq
                jrow = (keyb >> 6) & 63
                vals = plsc.load_gather(qd_v, [row, col], mask=msk)
                vals = jnp.where(is8, jnp.float32(1.0), vals)
                plsc.addupdate_scatter(comb_v, [i9, row, jrow], vals, mask=msk)
            return 0

        lax.fori_loop(0, (nq + 15) // 16, group_body, 0)

        pltpu.sync_copy(comb_v.at[pl.ds(0, _H)], corr_hbm.at[b])
        pltpu.sync_copy(comb_v.at[pl.ds(_H, 1)], cnt_hbm.at[b])


def _sc_corr(ekey, qdot):
    mesh = plsc.VectorSubcoreMesh(core_axis_name="c", subcore_axis_name="s")
    f = pl.kernel(
        _sc_body,
        out_type=(
            jax.ShapeDtypeStruct((_B, _H, _N, _NP), jnp.float32),
            jax.ShapeDtypeStruct((_B, 1, _N, _NP), jnp.float32),
        ),
        mesh=mesh,
        compiler_params=pltpu.CompilerParams(needs_layout_passes=False),
        scratch_types=[
            pltpu.VMEM((_CH,), jnp.int32),
            pltpu.VMEM((_QCAP,), jnp.int32),
            pltpu.VMEM((_N, _H * _NREL), jnp.float32),
            pltpu.VMEM((_H + 1, _N, _NP), jnp.float32),
        ],
    )
    return f(ekey, qdot)


def _attn_body(q_ref, k_ref, v_ref, corr_ref, cnt_ref, out_ref):
    qh = q_ref[0]
    kh = k_ref[0]
    vh = v_ref[0]
    corr = corr_ref[0][:, :, :_N]
    mask = (cnt_ref[0, 0][:, :_N] > 0.0)[None, :, :]
    scale = jnp.float32(1.0) / jnp.sqrt(jnp.float32(_D))
    neg = jnp.float32(-jnp.inf)
    lg = lax.dot_general(qh, kh, (((2,), (2,)), ((0,), (0,))),
                         preferred_element_type=jnp.float32)
    lg = (lg - corr) * scale
    ml = jnp.where(mask, lg, neg)
    m = jnp.max(ml, axis=2, keepdims=True)
    m = jnp.where(jnp.isfinite(m), m, 0.0)
    e = jnp.exp(jnp.where(mask, lg - m, jnp.float32(-1e30)))
    s = jnp.sum(e, axis=2, keepdims=True)
    p = jnp.where(s > 0, e / jnp.where(s > 0, s, 1.0), 0.0)
    o = lax.dot_general(p, vh, (((2,), (1,)), ((0,), (0,))),
                        preferred_element_type=jnp.float32)
    for h in range(_H):
        out_ref[0, :, h * _D:(h + 1) * _D] = o[h]


def _attn(q, k, v, corr, cnt):
    hd = pl.BlockSpec((1, _H, _N, _D), lambda b: (b, 0, 0, 0))
    return pl.pallas_call(
        _attn_body,
        grid=(_B,),
        in_specs=[
            hd, hd, hd,
            pl.BlockSpec((1, _H, _N, _NP), lambda b: (b, 0, 0, 0)),
            pl.BlockSpec((1, 1, _N, _NP), lambda b: (b, 0, 0, 0)),
        ],
        out_specs=pl.BlockSpec((1, _N, _HID), lambda b: (b, 0, 0)),
        out_shape=jax.ShapeDtypeStruct((_B, _N, _HID), jnp.float32),
    )(q, k, v, corr, cnt)


def kernel(node_states, edge_indices, node_type_ids, Wq, bq, Wk, bk, Wv, bv, edge_emb):
    x = node_states.reshape(_B * _N, _HID)
    q, k, v, qdot = _proj(x, Wq, bq.reshape(1, _HID), Wk, bk.reshape(1, _HID),
                          Wv, bv.reshape(1, _HID), edge_emb)
    ekey = _ekey(edge_indices)
    corr, cnt = _sc_corr(ekey, qdot)
    out = _attn(q, k, v, corr, cnt)
    return out
